# baseline (speedup 1.0000x reference)
import jax
import jax.numpy as jnp
from jax import lax
from jax.experimental import pallas as pl

_K = 24
_D = 12
_C = 25
_S = 32
_L = 65536


def _body(x_ref, w_ref, cn_ref, out_ref):
    x = x_ref[...]
    t = lax.dot_general(
        w_ref[...], x,
        dimension_numbers=(((1,), (0,)), ((), ())),
        preferred_element_type=jnp.float32,
        precision=lax.Precision.HIGHEST,
    )
    t = t - cn_ref[...]

    m = jnp.maximum(t[:16, :], t[16:, :])
    m = jnp.maximum(m[:8, :], m[8:, :])
    m = jnp.maximum(m[:4, :], m[4:, :])
    m = jnp.maximum(m[:2, :], m[2:, :])
    m = jnp.maximum(m[:1, :], m[1:, :])

    out_ref[...] = (t[:_C, :] == m).astype(jnp.float32)


def kernel(inputs, CTT):
    n = inputs.shape[0]
    grid = n // _L
    f32 = jnp.float32

    w = jnp.concatenate(
        [jnp.zeros((1, _D), f32), 2.0 * CTT, jnp.zeros((_S - 1 - _K, _D), f32)],
        axis=0,
    )
    cn = jnp.sum(CTT * CTT, axis=1)
    big = jnp.full((1,), 1e30, f32)
    cnp = jnp.concatenate([big, cn, jnp.full((_S - 1 - _K,), 1e30, f32)])[:, None]

    xt = inputs.T

    res = pl.pallas_call(
        _body,
        grid=(grid,),
        in_specs=[
            pl.BlockSpec((_D, _L), lambda i: (0, i)),
            pl.BlockSpec((_S, _D), lambda i: (0, 0)),
            pl.BlockSpec((_S, 1), lambda i: (0, 0)),
        ],
        out_specs=pl.BlockSpec((_C, _L), lambda i: (0, i)),
        out_shape=jax.ShapeDtypeStruct((_C, n), f32),
    )(xt, w, cnp)
    return res.T


# speedup vs baseline: 1.2984x; 1.2984x over previous
import jax
import jax.numpy as jnp
from jax import lax
from jax.experimental import pallas as pl

_K = 24
_D = 12
_C = 25
_S = 32
_L = 32768


def _body(x_ref, w_ref, cn_ref, out_ref):
    x = x_ref[...]
    bf16, f32 = jnp.bfloat16, jnp.float32
    xh = x.astype(bf16)
    r1 = x - xh.astype(f32)
    xm = r1.astype(bf16)
    xl = (r1 - xm.astype(f32)).astype(bf16)
    w = w_ref[...].astype(bf16)
    dn = (((1,), (0,)), ((), ()))
    t = (
        lax.dot_general(w, xh, dn, preferred_element_type=f32)
        + lax.dot_general(w, xm, dn, preferred_element_type=f32)
        + lax.dot_general(w, xl, dn, preferred_element_type=f32)
    )
    t = t - cn_ref[...]

    m = jnp.maximum(t[:16, :], t[16:, :])
    m = jnp.maximum(m[:8, :], m[8:, :])
    m = jnp.maximum(m[:4, :], m[4:, :])
    m = jnp.maximum(m[:2, :], m[2:, :])
    m = jnp.maximum(m[:1, :], m[1:, :])

    out_ref[...] = (t[:_C, :] == m).astype(jnp.float32)


def kernel(inputs, CTT):
    n = inputs.shape[0]
    grid = n // _L
    f32 = jnp.float32

    w = jnp.concatenate(
        [jnp.zeros((1, _D), f32), 2.0 * CTT, jnp.zeros((_S - 1 - _K, _D), f32)],
        axis=0,
    )
    cn = jnp.sum(CTT * CTT, axis=1)
    big = jnp.full((1,), 1e30, f32)
    cnp = jnp.concatenate([big, cn, jnp.full((_S - 1 - _K,), 1e30, f32)])[:, None]

    xt = inputs.T

    res = pl.pallas_call(
        _body,
        grid=(grid,),
        in_specs=[
            pl.BlockSpec((_D, _L), lambda i: (0, i)),
            pl.BlockSpec((_S, _D), lambda i: (0, 0)),
            pl.BlockSpec((_S, 1), lambda i: (0, 0)),
        ],
        out_specs=pl.BlockSpec((_C, _L), lambda i: (0, i)),
        out_shape=jax.ShapeDtypeStruct((_C, n), f32),
    )(xt, w, cnp)
    return res.T
